# baseline (speedup 1.0000x reference)
import functools

import jax
import jax.numpy as jnp
from jax import lax
from jax.experimental import pallas as pl
from jax.experimental.pallas import tpu as pltpu

_F32 = jnp.float32
_N = 32


def _fold_bn(w, bn, eps=1e-5):
    g, b, m, v = bn
    s = g / jnp.sqrt(v + eps)
    return w * s[:, None, None, None], b - m * s


def _conv3x3_mm(x0, x1, x2, wt, H, W, Wb, Cout):
    xr = jnp.concatenate([x0, x1, x2], axis=-1)
    z = jnp.dot(xr.reshape(H * Wb, xr.shape[-1]), wt,
                preferred_element_type=_F32)
    z = z.reshape(H, Wb, 3 * Cout)
    return (z[:, 0:W, 0:Cout]
            + z[:, 1:W + 1, Cout:2 * Cout]
            + z[:, 2:W + 2, 2 * Cout:3 * Cout])


def _store_padded(out_ref, o, H, W, Wb, C):
    out_ref[0] = jnp.zeros((H + 2, Wb, C), _F32)
    out_ref[0, 1:H + 1, 1:W + 1, :] = o


def _stem_kernel(x_ref, w_ref, b_ref, out_ref, s_ref):
    pieces = []
    for ky in range(7):
        r = x_ref[0, ky:ky + 223:2, :, :]
        for dj in range(2):
            pieces.append(r[:, dj:dj + 32, :])
    xr = jnp.concatenate(pieces, axis=-1)
    xrm = xr.reshape(112 * 32, 112)
    s_ref[...] = jnp.full((4, 114, 32, 64), -1e30, _F32)
    for q in range(4):
        z = jnp.dot(xrm, w_ref[q], preferred_element_type=_F32) + b_ref[...]
        s_ref[q, 1:113, 1:29, :] = z.reshape(112, 32, 64)[:, 0:28, :]
    m = []
    for q in range(4):
        a = s_ref[q, 0:111:2, :, :]
        b = s_ref[q, 1:112:2, :, :]
        c = s_ref[q, 2:113:2, :, :]
        m.append(jnp.maximum(jnp.maximum(a, b), c))
    p0 = jnp.maximum(jnp.maximum(m[3][:, 0:28], m[0][:, 1:29]), m[1][:, 1:29])
    p1 = jnp.maximum(jnp.maximum(m[1][:, 1:29], m[2][:, 1:29]), m[3][:, 1:29])
    p0 = jnp.maximum(p0, 0.0)
    p1 = jnp.maximum(p1, 0.0)
    out_ref[0] = jnp.zeros((58, 64, 64), _F32)
    out_ref[0, 1:57, 1:57:2, :] = p0
    out_ref[0, 1:57, 2:58:2, :] = p1


def _stem_call(x8, w, b):
    return pl.pallas_call(
        _stem_kernel,
        grid=(_N,),
        in_specs=[
            pl.BlockSpec((1, 232, 40, 8), lambda n: (n, 0, 0, 0)),
            pl.BlockSpec((4, 112, 64), lambda n: (0, 0, 0)),
            pl.BlockSpec((1, 64), lambda n: (0, 0)),
        ],
        out_specs=pl.BlockSpec((1, 58, 64, 64), lambda n: (n, 0, 0, 0)),
        out_shape=jax.ShapeDtypeStruct((_N, 58, 64, 64), _F32),
        scratch_shapes=[pltpu.VMEM((4, 114, 32, 64), _F32)],
    )(x8, w, b)


def _plain_block_kernel(x_ref, w1_ref, b1_ref, w2_ref, b2_ref, out_ref,
                        hp_ref, *, H, W, Wb, C):
    xp = x_ref[0]
    h = _conv3x3_mm(xp[0:H], xp[1:H + 1], xp[2:H + 2], w1_ref[...],
                    H, W, Wb, C) + b1_ref[...]
    h = jnp.maximum(h, 0.0)
    hp_ref[...] = jnp.zeros((H + 2, Wb, C), _F32)
    hp_ref[1:H + 1, 1:W + 1, :] = h
    o = _conv3x3_mm(hp_ref[0:H], hp_ref[1:H + 1], hp_ref[2:H + 2],
                    w2_ref[...], H, W, Wb, C) + b2_ref[...]
    o = jnp.maximum(o + xp[1:H + 1, 1:W + 1, :], 0.0)
    _store_padded(out_ref, o, H, W, Wb, C)


def _plain_block_call(buf, w1, b1, w2, b2, H, W, Wb, C):
    return pl.pallas_call(
        functools.partial(_plain_block_kernel, H=H, W=W, Wb=Wb, C=C),
        grid=(_N,),
        in_specs=[
            pl.BlockSpec((1, H + 2, Wb, C), lambda n: (n, 0, 0, 0)),
            pl.BlockSpec((3 * C, 3 * C), lambda n: (0, 0)),
            pl.BlockSpec((1, C), lambda n: (0, 0)),
            pl.BlockSpec((3 * C, 3 * C), lambda n: (0, 0)),
            pl.BlockSpec((1, C), lambda n: (0, 0)),
        ],
        out_specs=pl.BlockSpec((1, H + 2, Wb, C), lambda n: (n, 0, 0, 0)),
        out_shape=jax.ShapeDtypeStruct((_N, H + 2, Wb, C), _F32),
        scratch_shapes=[pltpu.VMEM((H + 2, Wb, C), _F32)],
    )(buf, w1, b1, w2, b2)


def _down_block_kernel(t_ref, xd_ref, w1_ref, b1_ref, w2_ref, b2_ref,
                       wd_ref, bd_ref, out_ref, hp_ref,
                       *, Ho, Wo, Wob, Wb2, Cin, Cout):
    t = t_ref[0]
    h = jnp.dot(t.reshape(Ho * Wob, 9 * Cin), w1_ref[...],
                preferred_element_type=_F32).reshape(Ho, Wob, Cout)
    h = jnp.maximum(h + b1_ref[...], 0.0)[:, 0:Wo, :]
    hp_ref[...] = jnp.zeros((Ho + 2, Wb2, Cout), _F32)
    hp_ref[1:Ho + 1, 1:Wo + 1, :] = h
    o = _conv3x3_mm(hp_ref[0:Ho], hp_ref[1:Ho + 1], hp_ref[2:Ho + 2],
                    w2_ref[...], Ho, Wo, Wb2, Cout) + b2_ref[...]
    res = jnp.dot(xd_ref[0].reshape(Ho * Wob, Cin), wd_ref[...],
                  preferred_element_type=_F32).reshape(Ho, Wob, Cout)
    res = res[:, 0:Wo, :] + bd_ref[...]
    o = jnp.maximum(o + res, 0.0)
    _store_padded(out_ref, o, Ho, Wo, Wb2, Cout)


def _s2_taps(buf, Ho, Wo, Wob):
    taps = [buf[:, ky:ky + 2 * Ho - 1:2, kx:kx + 2 * Wo - 1:2, :]
            for ky in range(3) for kx in range(3)]
    t = jnp.concatenate(taps, axis=-1)
    t = jnp.pad(t, ((0, 0), (0, 0), (0, Wob - Wo), (0, 0)))
    xds = buf[:, 1:2 * Ho:2, 1:2 * Wo:2, :]
    xds = jnp.pad(xds, ((0, 0), (0, 0), (0, Wob - Wo), (0, 0)))
    return t, xds


def _down_block_call(taps, xds, w1, b1, w2, b2, wd, bd,
                     Ho, Wo, Wob, Wb2, Cin, Cout):
    return pl.pallas_call(
        functools.partial(_down_block_kernel, Ho=Ho, Wo=Wo, Wob=Wob,
                          Wb2=Wb2, Cin=Cin, Cout=Cout),
        grid=(_N,),
        in_specs=[
            pl.BlockSpec((1, Ho, Wob, 9 * Cin), lambda n: (n, 0, 0, 0)),
            pl.BlockSpec((1, Ho, Wob, Cin), lambda n: (n, 0, 0, 0)),
            pl.BlockSpec((9 * Cin, Cout), lambda n: (0, 0)),
            pl.BlockSpec((1, Cout), lambda n: (0, 0)),
            pl.BlockSpec((3 * Cout, 3 * Cout), lambda n: (0, 0)),
            pl.BlockSpec((1, Cout), lambda n: (0, 0)),
            pl.BlockSpec((Cin, Cout), lambda n: (0, 0)),
            pl.BlockSpec((1, Cout), lambda n: (0, 0)),
        ],
        out_specs=pl.BlockSpec((1, Ho + 2, Wb2, Cout), lambda n: (n, 0, 0, 0)),
        out_shape=jax.ShapeDtypeStruct((_N, Ho + 2, Wb2, Cout), _F32),
        scratch_shapes=[pltpu.VMEM((Ho + 2, Wb2, Cout), _F32)],
    )(taps, xds, w1, b1, w2, b2, wd, bd)


def _head_kernel(x_ref, cw_ref, cb_ref, pw_ref, pb_ref, idx_ref, probs_ref):
    x = x_ref[...]
    feat = jnp.sum(x, axis=(1, 2)) * (1.0 / 49.0)
    cl = jnp.dot(feat, cw_ref[...], preferred_element_type=_F32) + cb_ref[...]
    pll = jnp.dot(feat, pw_ref[...], preferred_element_type=_F32) + pb_ref[...]

    def smax(l):
        m = jnp.max(l, axis=1, keepdims=True)
        e = jnp.exp(l - m)
        return e / jnp.sum(e, axis=1, keepdims=True)

    cp = smax(cl)
    pp = smax(pll)
    cmax = jnp.max(cp, axis=1, keepdims=True)
    pmax = jnp.max(pp, axis=1, keepdims=True)
    ciota = lax.broadcasted_iota(jnp.int32, (_N, 6), 1)
    piota = lax.broadcasted_iota(jnp.int32, (_N, 2), 1)
    cidx = jnp.min(jnp.where(cp == cmax, ciota, 6), axis=1, keepdims=True)
    pidx = jnp.min(jnp.where(pp == pmax, piota, 2), axis=1, keepdims=True)
    wp = 0.3 * pmax
    wc = 0.7 * cmax
    tot = wp + wc + 1e-8
    idx_ref[...] = jnp.concatenate([pidx, cidx + 2], axis=1)
    probs_ref[...] = jnp.concatenate([wp / tot, wc / tot], axis=1)


def _head_call(buf4, cw, cb, pw, pb):
    return pl.pallas_call(
        _head_kernel,
        out_shape=[jax.ShapeDtypeStruct((_N, 2), jnp.int32),
                   jax.ShapeDtypeStruct((_N, 2), _F32)],
    )(buf4, cw, cb, pw, pb)


def _w3x3(w, bn):
    w, b = _fold_bn(w, bn)
    wt = jnp.transpose(w, (2, 1, 3, 0))
    C, Cout = w.shape[1], w.shape[0]
    return wt.reshape(3 * C, 3 * Cout), b.reshape(1, Cout)


def _w3x3s2(w, bn):
    w, b = _fold_bn(w, bn)
    wt = jnp.transpose(w, (2, 3, 1, 0))
    return wt.reshape(9 * w.shape[1], w.shape[0]), b.reshape(1, w.shape[0])


def _w1x1(w, bn):
    w, b = _fold_bn(w, bn)
    return jnp.transpose(w[:, :, 0, 0], (1, 0)), b.reshape(1, w.shape[0])


def _wstem(w, bn):
    w, b = _fold_bn(w, bn)
    wk = w[:, 0]
    z = jnp.zeros((64,), _F32)
    qmats = []
    for q in range(4):
        rows = []
        for ky in range(7):
            for dj in range(2):
                for l in range(8):
                    kx = 8 * dj + l - 2 * q
                    rows.append(wk[:, ky, kx] if 0 <= kx < 7 else z)
        qmats.append(jnp.stack(rows, axis=0))
    return jnp.stack(qmats, axis=0), b.reshape(1, 64)


_GEOM = [
    (56, 56, 64), (28, 28, 32), (14, 14, 16), (7, 7, 16),
]
_CH = [64, 128, 256, 512]


def kernel(x, params):
    xp = jnp.pad(x[:, 0], ((0, 0), (3, 5), (3, 85)))
    xp = jnp.pad(xp, ((0, 0), (0, 0), (0, 8)))
    x8 = xp.reshape(_N, 232, 40, 8)
    ws, bs = _wstem(params['conv1'], params['bn1'])
    buf = _stem_call(x8, ws, bs)

    for li, layer in enumerate(params['layers']):
        H, W, Wb = _GEOM[li]
        C = _CH[li]
        blk0, blk1 = layer
        if li == 0:
            w1, b1 = _w3x3(blk0['conv1'], blk0['bn1'])
            w2, b2 = _w3x3(blk0['conv2'], blk0['bn2'])
            buf = _plain_block_call(buf, w1, b1, w2, b2, H, W, Wb, C)
        else:
            Cin = _CH[li - 1]
            Wob = 8 * ((W + 7) // 8)
            w1, b1 = _w3x3s2(blk0['conv1'], blk0['bn1'])
            w2, b2 = _w3x3(blk0['conv2'], blk0['bn2'])
            wd, bd = _w1x1(blk0['down'], blk0['dbn'])
            taps, xds = _s2_taps(buf, H, W, Wob)
            buf = _down_block_call(taps, xds, w1, b1, w2, b2, wd, bd,
                                   H, W, Wob, Wb, Cin, C)
        w1, b1 = _w3x3(blk1['conv1'], blk1['bn1'])
        w2, b2 = _w3x3(blk1['conv2'], blk1['bn2'])
        buf = _plain_block_call(buf, w1, b1, w2, b2, H, W, Wb, C)

    idx, probs = _head_call(
        buf,
        jnp.transpose(params['child_w'], (1, 0)),
        params['child_b'].reshape(1, 6),
        jnp.transpose(params['parent_w'], (1, 0)),
        params['parent_b'].reshape(1, 2),
    )
    return (idx, probs)


# speedup vs baseline: 1.0045x; 1.0045x over previous
import functools

import jax
import jax.numpy as jnp
from jax import lax
from jax.experimental import pallas as pl
from jax.experimental.pallas import tpu as pltpu

_F32 = jnp.float32
_N = 32


def _fold_bn(w, bn, eps=1e-5):
    g, b, m, v = bn
    s = g / jnp.sqrt(v + eps)
    return w * s[:, None, None, None], b - m * s


def _conv3x3_mm(x0, x1, x2, wt, B, H, W, Wb, Cout):
    xr = jnp.concatenate([x0, x1, x2], axis=-1)
    z = jnp.dot(xr.reshape(B * H * Wb, xr.shape[-1]), wt,
                preferred_element_type=_F32)
    z = z.reshape(B, H, Wb, 3 * Cout)
    return (z[:, :, 0:W, 0:Cout]
            + z[:, :, 1:W + 1, Cout:2 * Cout]
            + z[:, :, 2:W + 2, 2 * Cout:3 * Cout])


def _store_padded(out_ref, o, B, H, W, Wb, C):
    out_ref[...] = jnp.zeros((B, H + 2, Wb, C), _F32)
    out_ref[:, 1:H + 1, 1:W + 1, :] = o


def _stem_kernel(x_ref, w_ref, b_ref, out_ref, s_ref):
    pieces = []
    for ky in range(7):
        r = x_ref[0, ky:ky + 223:2, :, :]
        for dj in range(2):
            pieces.append(r[:, dj:dj + 32, :])
    xr = jnp.concatenate(pieces, axis=-1)
    xrm = xr.reshape(112 * 32, 112)
    z = jnp.dot(xrm, w_ref[...], preferred_element_type=_F32)
    z = z.reshape(112, 32, 256)
    s_ref[...] = jnp.full((4, 114, 32, 64), -1e30, _F32)
    for q in range(4):
        s_ref[q, 1:113, 1:29, :] = z[:, 0:28, 64 * q:64 * q + 64] + b_ref[...]
    m = []
    for q in range(4):
        a = s_ref[q, 0:111:2, :, :]
        b = s_ref[q, 1:112:2, :, :]
        c = s_ref[q, 2:113:2, :, :]
        m.append(jnp.maximum(jnp.maximum(a, b), c))
    p0 = jnp.maximum(jnp.maximum(m[3][:, 0:28], m[0][:, 1:29]), m[1][:, 1:29])
    p1 = jnp.maximum(jnp.maximum(m[1][:, 1:29], m[2][:, 1:29]), m[3][:, 1:29])
    p0 = jnp.maximum(p0, 0.0)
    p1 = jnp.maximum(p1, 0.0)
    out_ref[0] = jnp.zeros((58, 64, 64), _F32)
    out_ref[0, 1:57, 1:57:2, :] = p0
    out_ref[0, 1:57, 2:58:2, :] = p1


def _stem_call(x8, w, b):
    return pl.pallas_call(
        _stem_kernel,
        grid=(_N,),
        in_specs=[
            pl.BlockSpec((1, 232, 40, 8), lambda n: (n, 0, 0, 0)),
            pl.BlockSpec((112, 256), lambda n: (0, 0)),
            pl.BlockSpec((1, 64), lambda n: (0, 0)),
        ],
        out_specs=pl.BlockSpec((1, 58, 64, 64), lambda n: (n, 0, 0, 0)),
        out_shape=jax.ShapeDtypeStruct((_N, 58, 64, 64), _F32),
        scratch_shapes=[pltpu.VMEM((4, 114, 32, 64), _F32)],
    )(x8, w, b)


def _plain_block_kernel(x_ref, w1_ref, b1_ref, w2_ref, b2_ref, out_ref,
                        hp_ref, *, B, H, W, Wb, C):
    xp = x_ref[...]
    h = _conv3x3_mm(xp[:, 0:H], xp[:, 1:H + 1], xp[:, 2:H + 2], w1_ref[...],
                    B, H, W, Wb, C) + b1_ref[...]
    h = jnp.maximum(h, 0.0)
    hp_ref[...] = jnp.zeros((B, H + 2, Wb, C), _F32)
    hp_ref[:, 1:H + 1, 1:W + 1, :] = h
    o = _conv3x3_mm(hp_ref[:, 0:H], hp_ref[:, 1:H + 1], hp_ref[:, 2:H + 2],
                    w2_ref[...], B, H, W, Wb, C) + b2_ref[...]
    o = jnp.maximum(o + xp[:, 1:H + 1, 1:W + 1, :], 0.0)
    _store_padded(out_ref, o, B, H, W, Wb, C)


def _plain_block_call(buf, w1, b1, w2, b2, B, H, W, Wb, C):
    return pl.pallas_call(
        functools.partial(_plain_block_kernel, B=B, H=H, W=W, Wb=Wb, C=C),
        grid=(_N // B,),
        in_specs=[
            pl.BlockSpec((B, H + 2, Wb, C), lambda n: (n, 0, 0, 0)),
            pl.BlockSpec((3 * C, 3 * C), lambda n: (0, 0)),
            pl.BlockSpec((1, C), lambda n: (0, 0)),
            pl.BlockSpec((3 * C, 3 * C), lambda n: (0, 0)),
            pl.BlockSpec((1, C), lambda n: (0, 0)),
        ],
        out_specs=pl.BlockSpec((B, H + 2, Wb, C), lambda n: (n, 0, 0, 0)),
        out_shape=jax.ShapeDtypeStruct((_N, H + 2, Wb, C), _F32),
        scratch_shapes=[pltpu.VMEM((B, H + 2, Wb, C), _F32)],
    )(buf, w1, b1, w2, b2)


def _down_block_kernel(t_ref, xd_ref, w1_ref, b1_ref, w2_ref, b2_ref,
                       wd_ref, bd_ref, out_ref, hp_ref,
                       *, B, Ho, Wo, Wob, Wb2, Cin, Cout):
    t = t_ref[...]
    h = jnp.dot(t.reshape(B * Ho * Wob, 9 * Cin), w1_ref[...],
                preferred_element_type=_F32).reshape(B, Ho, Wob, Cout)
    h = jnp.maximum(h + b1_ref[...], 0.0)[:, :, 0:Wo, :]
    hp_ref[...] = jnp.zeros((B, Ho + 2, Wb2, Cout), _F32)
    hp_ref[:, 1:Ho + 1, 1:Wo + 1, :] = h
    o = _conv3x3_mm(hp_ref[:, 0:Ho], hp_ref[:, 1:Ho + 1], hp_ref[:, 2:Ho + 2],
                    w2_ref[...], B, Ho, Wo, Wb2, Cout) + b2_ref[...]
    res = jnp.dot(xd_ref[...].reshape(B * Ho * Wob, Cin), wd_ref[...],
                  preferred_element_type=_F32).reshape(B, Ho, Wob, Cout)
    res = res[:, :, 0:Wo, :] + bd_ref[...]
    o = jnp.maximum(o + res, 0.0)
    _store_padded(out_ref, o, B, Ho, Wo, Wb2, Cout)


def _s2_taps(buf, Ho, Wo, Wob):
    taps = [buf[:, ky:ky + 2 * Ho - 1:2, kx:kx + 2 * Wo - 1:2, :]
            for ky in range(3) for kx in range(3)]
    t = jnp.concatenate(taps, axis=-1)
    t = jnp.pad(t, ((0, 0), (0, 0), (0, Wob - Wo), (0, 0)))
    xds = buf[:, 1:2 * Ho:2, 1:2 * Wo:2, :]
    xds = jnp.pad(xds, ((0, 0), (0, 0), (0, Wob - Wo), (0, 0)))
    return t, xds


def _down_block_call(taps, xds, w1, b1, w2, b2, wd, bd,
                     B, Ho, Wo, Wob, Wb2, Cin, Cout):
    return pl.pallas_call(
        functools.partial(_down_block_kernel, B=B, Ho=Ho, Wo=Wo, Wob=Wob,
                          Wb2=Wb2, Cin=Cin, Cout=Cout),
        grid=(_N // B,),
        in_specs=[
            pl.BlockSpec((B, Ho, Wob, 9 * Cin), lambda n: (n, 0, 0, 0)),
            pl.BlockSpec((B, Ho, Wob, Cin), lambda n: (n, 0, 0, 0)),
            pl.BlockSpec((9 * Cin, Cout), lambda n: (0, 0)),
            pl.BlockSpec((1, Cout), lambda n: (0, 0)),
            pl.BlockSpec((3 * Cout, 3 * Cout), lambda n: (0, 0)),
            pl.BlockSpec((1, Cout), lambda n: (0, 0)),
            pl.BlockSpec((Cin, Cout), lambda n: (0, 0)),
            pl.BlockSpec((1, Cout), lambda n: (0, 0)),
        ],
        out_specs=pl.BlockSpec((B, Ho + 2, Wb2, Cout), lambda n: (n, 0, 0, 0)),
        out_shape=jax.ShapeDtypeStruct((_N, Ho + 2, Wb2, Cout), _F32),
        scratch_shapes=[pltpu.VMEM((B, Ho + 2, Wb2, Cout), _F32)],
    )(taps, xds, w1, b1, w2, b2, wd, bd)


def _head_kernel(x_ref, cw_ref, cb_ref, pw_ref, pb_ref, idx_ref, probs_ref):
    x = x_ref[...]
    feat = jnp.sum(x, axis=(1, 2)) * (1.0 / 49.0)
    cl = jnp.dot(feat, cw_ref[...], preferred_element_type=_F32) + cb_ref[...]
    pll = jnp.dot(feat, pw_ref[...], preferred_element_type=_F32) + pb_ref[...]

    def smax(l):
        m = jnp.max(l, axis=1, keepdims=True)
        e = jnp.exp(l - m)
        return e / jnp.sum(e, axis=1, keepdims=True)

    cp = smax(cl)
    pp = smax(pll)
    cmax = jnp.max(cp, axis=1, keepdims=True)
    pmax = jnp.max(pp, axis=1, keepdims=True)
    ciota = lax.broadcasted_iota(jnp.int32, (_N, 6), 1)
    piota = lax.broadcasted_iota(jnp.int32, (_N, 2), 1)
    cidx = jnp.min(jnp.where(cp == cmax, ciota, 6), axis=1, keepdims=True)
    pidx = jnp.min(jnp.where(pp == pmax, piota, 2), axis=1, keepdims=True)
    wp = 0.3 * pmax
    wc = 0.7 * cmax
    tot = wp + wc + 1e-8
    idx_ref[...] = jnp.concatenate([pidx, cidx + 2], axis=1)
    probs_ref[...] = jnp.concatenate([wp / tot, wc / tot], axis=1)


def _head_call(buf4, cw, cb, pw, pb):
    return pl.pallas_call(
        _head_kernel,
        out_shape=[jax.ShapeDtypeStruct((_N, 2), jnp.int32),
                   jax.ShapeDtypeStruct((_N, 2), _F32)],
    )(buf4, cw, cb, pw, pb)


def _w3x3(w, bn):
    w, b = _fold_bn(w, bn)
    wt = jnp.transpose(w, (2, 1, 3, 0))
    C, Cout = w.shape[1], w.shape[0]
    return wt.reshape(3 * C, 3 * Cout), b.reshape(1, Cout)


def _w3x3s2(w, bn):
    w, b = _fold_bn(w, bn)
    wt = jnp.transpose(w, (2, 3, 1, 0))
    return wt.reshape(9 * w.shape[1], w.shape[0]), b.reshape(1, w.shape[0])


def _w1x1(w, bn):
    w, b = _fold_bn(w, bn)
    return jnp.transpose(w[:, :, 0, 0], (1, 0)), b.reshape(1, w.shape[0])


def _wstem(w, bn):
    w, b = _fold_bn(w, bn)
    wk = w[:, 0]
    z = jnp.zeros((64,), _F32)
    qmats = []
    for q in range(4):
        rows = []
        for ky in range(7):
            for dj in range(2):
                for l in range(8):
                    kx = 8 * dj + l - 2 * q
                    rows.append(wk[:, ky, kx] if 0 <= kx < 7 else z)
        qmats.append(jnp.stack(rows, axis=0))
    return jnp.concatenate(qmats, axis=1), b.reshape(1, 64)


_GEOM = [
    (56, 56, 64), (28, 28, 32), (14, 14, 16), (7, 7, 16),
]
_CH = [64, 128, 256, 512]
_BB = [2, 4, 8, 8]


def kernel(x, params):
    xp = jnp.pad(x[:, 0], ((0, 0), (3, 5), (3, 85)))
    xp = jnp.pad(xp, ((0, 0), (0, 0), (0, 8)))
    x8 = xp.reshape(_N, 232, 40, 8)
    ws, bs = _wstem(params['conv1'], params['bn1'])
    buf = _stem_call(x8, ws, bs)

    for li, layer in enumerate(params['layers']):
        H, W, Wb = _GEOM[li]
        C = _CH[li]
        B = _BB[li]
        blk0, blk1 = layer
        if li == 0:
            w1, b1 = _w3x3(blk0['conv1'], blk0['bn1'])
            w2, b2 = _w3x3(blk0['conv2'], blk0['bn2'])
            buf = _plain_block_call(buf, w1, b1, w2, b2, B, H, W, Wb, C)
        else:
            Cin = _CH[li - 1]
            Wob = 8 * ((W + 7) // 8)
            w1, b1 = _w3x3s2(blk0['conv1'], blk0['bn1'])
            w2, b2 = _w3x3(blk0['conv2'], blk0['bn2'])
            wd, bd = _w1x1(blk0['down'], blk0['dbn'])
            taps, xds = _s2_taps(buf, H, W, Wob)
            buf = _down_block_call(taps, xds, w1, b1, w2, b2, wd, bd,
                                   B, H, W, Wob, Wb, Cin, C)
        w1, b1 = _w3x3(blk1['conv1'], blk1['bn1'])
        w2, b2 = _w3x3(blk1['conv2'], blk1['bn2'])
        buf = _plain_block_call(buf, w1, b1, w2, b2, B, H, W, Wb, C)

    idx, probs = _head_call(
        buf,
        jnp.transpose(params['child_w'], (1, 0)),
        params['child_b'].reshape(1, 6),
        jnp.transpose(params['parent_w'], (1, 0)),
        params['parent_b'].reshape(1, 2),
    )
    return (idx, probs)


# speedup vs baseline: 8.3050x; 8.2677x over previous
import functools

import jax
import jax.numpy as jnp
from jax import lax
from jax.experimental import pallas as pl
from jax.experimental.pallas import tpu as pltpu

_F32 = jnp.float32
_N = 32


def _fold_bn(w, bn, eps=1e-5):
    g, b, m, v = bn
    s = g / jnp.sqrt(v + eps)
    return w * s[:, None, None, None], b - m * s


def _conv3x3_mm(x0, x1, x2, wt, B, H, W, Wb, Cout):
    xr = jnp.concatenate([x0, x1, x2], axis=-1)
    z = jnp.dot(xr.reshape(B * H * Wb, xr.shape[-1]), wt,
                preferred_element_type=_F32)
    z = z.reshape(B, H, Wb, 3 * Cout)
    return (z[:, :, 0:W, 0:Cout]
            + z[:, :, 1:W + 1, Cout:2 * Cout]
            + z[:, :, 2:W + 2, 2 * Cout:3 * Cout])


def _store_padded(out_ref, o, B, H, W, Wb, C):
    out_ref[...] = jnp.zeros((B, H + 2, Wb, C), _F32)
    out_ref[:, 1:H + 1, 1:W + 1, :] = o


def _stem_kernel(x_ref, w_ref, b_ref, out_ref, s_ref):
    pieces = []
    for ky in range(7):
        r = x_ref[0, ky:ky + 223:2, :, :]
        for dj in range(2):
            pieces.append(r[:, dj:dj + 32, :])
    xr = jnp.concatenate(pieces, axis=-1)
    xrm = xr.reshape(112 * 32, 112)
    z = jnp.dot(xrm, w_ref[...], preferred_element_type=_F32)
    z = z.reshape(112, 32, 256)
    s_ref[...] = jnp.full((4, 114, 32, 64), -1e30, _F32)
    for q in range(4):
        s_ref[q, 1:113, 1:29, :] = z[:, 0:28, 64 * q:64 * q + 64] + b_ref[...]
    m = []
    for q in range(4):
        a = s_ref[q, 0:111:2, :, :]
        b = s_ref[q, 1:112:2, :, :]
        c = s_ref[q, 2:113:2, :, :]
        m.append(jnp.maximum(jnp.maximum(a, b), c))
    p0 = jnp.maximum(jnp.maximum(m[3][:, 0:28], m[0][:, 1:29]), m[1][:, 1:29])
    p1 = jnp.maximum(jnp.maximum(m[1][:, 1:29], m[2][:, 1:29]), m[3][:, 1:29])
    p0 = jnp.maximum(p0, 0.0)
    p1 = jnp.maximum(p1, 0.0)
    out_ref[0] = jnp.zeros((58, 64, 64), _F32)
    out_ref[0, 1:57, 1:57:2, :] = p0
    out_ref[0, 1:57, 2:58:2, :] = p1


def _stem_call(x8, w, b):
    return pl.pallas_call(
        _stem_kernel,
        grid=(_N,),
        in_specs=[
            pl.BlockSpec((1, 232, 40, 8), lambda n: (n, 0, 0, 0)),
            pl.BlockSpec((112, 256), lambda n: (0, 0)),
            pl.BlockSpec((1, 64), lambda n: (0, 0)),
        ],
        out_specs=pl.BlockSpec((1, 58, 64, 64), lambda n: (n, 0, 0, 0)),
        out_shape=jax.ShapeDtypeStruct((_N, 58, 64, 64), _F32),
        scratch_shapes=[pltpu.VMEM((4, 114, 32, 64), _F32)],
    )(x8, w, b)


def _plain_block_kernel(x_ref, w1_ref, b1_ref, w2_ref, b2_ref, out_ref,
                        hp_ref, *, B, H, W, Wb, C):
    xp = x_ref[...]
    h = _conv3x3_mm(xp[:, 0:H], xp[:, 1:H + 1], xp[:, 2:H + 2], w1_ref[...],
                    B, H, W, Wb, C) + b1_ref[...]
    h = jnp.maximum(h, 0.0)
    hp_ref[...] = jnp.zeros((B, H + 2, Wb, C), _F32)
    hp_ref[:, 1:H + 1, 1:W + 1, :] = h
    o = _conv3x3_mm(hp_ref[:, 0:H], hp_ref[:, 1:H + 1], hp_ref[:, 2:H + 2],
                    w2_ref[...], B, H, W, Wb, C) + b2_ref[...]
    o = jnp.maximum(o + xp[:, 1:H + 1, 1:W + 1, :], 0.0)
    _store_padded(out_ref, o, B, H, W, Wb, C)


def _plain_block_call(buf, w1, b1, w2, b2, B, H, W, Wb, C):
    return pl.pallas_call(
        functools.partial(_plain_block_kernel, B=B, H=H, W=W, Wb=Wb, C=C),
        grid=(_N // B,),
        in_specs=[
            pl.BlockSpec((B, H + 2, Wb, C), lambda n: (n, 0, 0, 0)),
            pl.BlockSpec((3 * C, 3 * C), lambda n: (0, 0)),
            pl.BlockSpec((1, C), lambda n: (0, 0)),
            pl.BlockSpec((3 * C, 3 * C), lambda n: (0, 0)),
            pl.BlockSpec((1, C), lambda n: (0, 0)),
        ],
        out_specs=pl.BlockSpec((B, H + 2, Wb, C), lambda n: (n, 0, 0, 0)),
        out_shape=jax.ShapeDtypeStruct((_N, H + 2, Wb, C), _F32),
        scratch_shapes=[pltpu.VMEM((B, H + 2, Wb, C), _F32)],
    )(buf, w1, b1, w2, b2)


def _down_block_kernel(p_ref, w1_ref, b1_ref, w2_ref, b2_ref,
                       wd_ref, bd_ref, out_ref, hp_ref,
                       *, B, Ho, Wo, Wob, Wb2, Cin, Cout):
    taps = [p_ref[:, 2 * (ky % 2) + (kx % 2),
                  ky // 2:ky // 2 + Ho, kx // 2:kx // 2 + Wo, :]
            for ky in range(3) for kx in range(3)]
    t = jnp.concatenate(taps, axis=-1)
    if Wob > Wo:
        t = jnp.concatenate(
            [t, jnp.zeros((B, Ho, Wob - Wo, 9 * Cin), _F32)], axis=2)
    h = jnp.dot(t.reshape(B * Ho * Wob, 9 * Cin), w1_ref[...],
                preferred_element_type=_F32).reshape(B, Ho, Wob, Cout)
    h = jnp.maximum(h + b1_ref[...], 0.0)[:, :, 0:Wo, :]
    hp_ref[...] = jnp.zeros((B, Ho + 2, Wb2, Cout), _F32)
    hp_ref[:, 1:Ho + 1, 1:Wo + 1, :] = h
    o = _conv3x3_mm(hp_ref[:, 0:Ho], hp_ref[:, 1:Ho + 1], hp_ref[:, 2:Ho + 2],
                    w2_ref[...], B, Ho, Wo, Wb2, Cout) + b2_ref[...]
    xd = p_ref[:, 3, 0:Ho, 0:Wo, :]
    if Wob > Wo:
        xd = jnp.concatenate(
            [xd, jnp.zeros((B, Ho, Wob - Wo, Cin), _F32)], axis=2)
    res = jnp.dot(xd.reshape(B * Ho * Wob, Cin), wd_ref[...],
                  preferred_element_type=_F32).reshape(B, Ho, Wob, Cout)
    res = res[:, :, 0:Wo, :] + bd_ref[...]
    o = jnp.maximum(o + res, 0.0)
    _store_padded(out_ref, o, B, Ho, Wo, Wb2, Cout)


def _parity4(buf):
    N, Hp, Wb, C = buf.shape
    v = buf.reshape(N, Hp // 2, 2, Wb // 2, 2, C)
    v = jnp.transpose(v, (0, 2, 4, 1, 3, 5))
    return v.reshape(N, 4, Hp // 2, Wb // 2, C)


def _down_block_call(pt, w1, b1, w2, b2, wd, bd,
                     B, Ho, Wo, Wob, Wb2, Cin, Cout):
    _, _, Hq, Wq, _ = pt.shape
    return pl.pallas_call(
        functools.partial(_down_block_kernel, B=B, Ho=Ho, Wo=Wo, Wob=Wob,
                          Wb2=Wb2, Cin=Cin, Cout=Cout),
        grid=(_N // B,),
        in_specs=[
            pl.BlockSpec((B, 4, Hq, Wq, Cin), lambda n: (n, 0, 0, 0, 0)),
            pl.BlockSpec((9 * Cin, Cout), lambda n: (0, 0)),
            pl.BlockSpec((1, Cout), lambda n: (0, 0)),
            pl.BlockSpec((3 * Cout, 3 * Cout), lambda n: (0, 0)),
            pl.BlockSpec((1, Cout), lambda n: (0, 0)),
            pl.BlockSpec((Cin, Cout), lambda n: (0, 0)),
            pl.BlockSpec((1, Cout), lambda n: (0, 0)),
        ],
        out_specs=pl.BlockSpec((B, Ho + 2, Wb2, Cout), lambda n: (n, 0, 0, 0)),
        out_shape=jax.ShapeDtypeStruct((_N, Ho + 2, Wb2, Cout), _F32),
        scratch_shapes=[pltpu.VMEM((B, Ho + 2, Wb2, Cout), _F32)],
    )(pt, w1, b1, w2, b2, wd, bd)


def _head_kernel(x_ref, cw_ref, cb_ref, pw_ref, pb_ref, idx_ref, probs_ref):
    x = x_ref[...]
    feat = jnp.sum(x, axis=(1, 2)) * (1.0 / 49.0)
    cl = jnp.dot(feat, cw_ref[...], preferred_element_type=_F32) + cb_ref[...]
    pll = jnp.dot(feat, pw_ref[...], preferred_element_type=_F32) + pb_ref[...]

    def smax(l):
        m = jnp.max(l, axis=1, keepdims=True)
        e = jnp.exp(l - m)
        return e / jnp.sum(e, axis=1, keepdims=True)

    cp = smax(cl)
    pp = smax(pll)
    cmax = jnp.max(cp, axis=1, keepdims=True)
    pmax = jnp.max(pp, axis=1, keepdims=True)
    ciota = lax.broadcasted_iota(jnp.int32, (_N, 6), 1)
    piota = lax.broadcasted_iota(jnp.int32, (_N, 2), 1)
    cidx = jnp.min(jnp.where(cp == cmax, ciota, 6), axis=1, keepdims=True)
    pidx = jnp.min(jnp.where(pp == pmax, piota, 2), axis=1, keepdims=True)
    wp = 0.3 * pmax
    wc = 0.7 * cmax
    tot = wp + wc + 1e-8
    idx_ref[...] = jnp.concatenate([pidx, cidx + 2], axis=1)
    probs_ref[...] = jnp.concatenate([wp / tot, wc / tot], axis=1)


def _head_call(buf4, cw, cb, pw, pb):
    return pl.pallas_call(
        _head_kernel,
        out_shape=[jax.ShapeDtypeStruct((_N, 2), jnp.int32),
                   jax.ShapeDtypeStruct((_N, 2), _F32)],
    )(buf4, cw, cb, pw, pb)


def _w3x3(w, bn):
    w, b = _fold_bn(w, bn)
    wt = jnp.transpose(w, (2, 1, 3, 0))
    C, Cout = w.shape[1], w.shape[0]
    return wt.reshape(3 * C, 3 * Cout), b.reshape(1, Cout)


def _w3x3s2(w, bn):
    w, b = _fold_bn(w, bn)
    wt = jnp.transpose(w, (2, 3, 1, 0))
    return wt.reshape(9 * w.shape[1], w.shape[0]), b.reshape(1, w.shape[0])


def _w1x1(w, bn):
    w, b = _fold_bn(w, bn)
    return jnp.transpose(w[:, :, 0, 0], (1, 0)), b.reshape(1, w.shape[0])


def _wstem(w, bn):
    w, b = _fold_bn(w, bn)
    wk = w[:, 0]
    z = jnp.zeros((64,), _F32)
    qmats = []
    for q in range(4):
        rows = []
        for ky in range(7):
            for dj in range(2):
                for l in range(8):
                    kx = 8 * dj + l - 2 * q
                    rows.append(wk[:, ky, kx] if 0 <= kx < 7 else z)
        qmats.append(jnp.stack(rows, axis=0))
    return jnp.concatenate(qmats, axis=1), b.reshape(1, 64)


_GEOM = [
    (56, 56, 64), (28, 28, 32), (14, 14, 16), (7, 7, 16),
]
_CH = [64, 128, 256, 512]
_BB = [2, 4, 8, 8]


def kernel(x, params):
    xp = jnp.pad(x[:, 0], ((0, 0), (3, 5), (3, 85)))
    xp = jnp.pad(xp, ((0, 0), (0, 0), (0, 8)))
    x8 = xp.reshape(_N, 232, 40, 8)
    ws, bs = _wstem(params['conv1'], params['bn1'])
    buf = _stem_call(x8, ws, bs)

    for li, layer in enumerate(params['layers']):
        H, W, Wb = _GEOM[li]
        C = _CH[li]
        B = _BB[li]
        blk0, blk1 = layer
        if li == 0:
            w1, b1 = _w3x3(blk0['conv1'], blk0['bn1'])
            w2, b2 = _w3x3(blk0['conv2'], blk0['bn2'])
            buf = _plain_block_call(buf, w1, b1, w2, b2, B, H, W, Wb, C)
        else:
            Cin = _CH[li - 1]
            Wob = 8 * ((W + 7) // 8)
            w1, b1 = _w3x3s2(blk0['conv1'], blk0['bn1'])
            w2, b2 = _w3x3(blk0['conv2'], blk0['bn2'])
            wd, bd = _w1x1(blk0['down'], blk0['dbn'])
            buf = _down_block_call(_parity4(buf), w1, b1, w2, b2, wd, bd,
                                   B, H, W, Wob, Wb, Cin, C)
        w1, b1 = _w3x3(blk1['conv1'], blk1['bn1'])
        w2, b2 = _w3x3(blk1['conv2'], blk1['bn2'])
        buf = _plain_block_call(buf, w1, b1, w2, b2, B, H, W, Wb, C)

    idx, probs = _head_call(
        buf,
        jnp.transpose(params['child_w'], (1, 0)),
        params['child_b'].reshape(1, 6),
        jnp.transpose(params['parent_w'], (1, 0)),
        params['parent_b'].reshape(1, 2),
    )
    return (idx, probs)


# speedup vs baseline: 8.5493x; 1.0294x over previous
import functools

import jax
import jax.numpy as jnp
from jax import lax
from jax.experimental import pallas as pl
from jax.experimental.pallas import tpu as pltpu

_F32 = jnp.float32
_N = 32


def _fold_bn(w, bn, eps=1e-5):
    g, b, m, v = bn
    s = g / jnp.sqrt(v + eps)
    return w * s[:, None, None, None], b - m * s


def _conv3x3_mm(x0, x1, x2, wt, B, H, W, Wb, Cout):
    xr = jnp.concatenate([x0, x1, x2], axis=-1)
    z = jnp.dot(xr.reshape(B * H * Wb, xr.shape[-1]), wt,
                preferred_element_type=_F32)
    z = z.reshape(B, H, Wb, 3 * Cout)
    return (z[:, :, 0:W, 0:Cout]
            + z[:, :, 1:W + 1, Cout:2 * Cout]
            + z[:, :, 2:W + 2, 2 * Cout:3 * Cout])


def _store_padded(out_ref, o, B, H, W, Wb, C):
    out_ref[...] = jnp.zeros((B, H + 2, Wb, C), _F32)
    out_ref[:, 1:H + 1, 1:W + 1, :] = o


def _stem_kernel(x_ref, w_ref, b_ref, out_ref, s_ref, *, B):
    pieces = []
    for ky in range(7):
        r = x_ref[:, ky % 2, ky // 2:ky // 2 + 112, :, :]
        for dj in range(2):
            pieces.append(r[:, :, dj:dj + 32, :])
    xr = jnp.concatenate(pieces, axis=-1)
    xrm = xr.reshape(B * 112 * 32, 112)
    z = jnp.dot(xrm, w_ref[...], preferred_element_type=_F32)
    z = z.reshape(B, 112, 32, 256)
    s_ref[...] = jnp.full((B, 4, 114, 32, 64), -1e30, _F32)
    for q in range(4):
        s_ref[:, q, 1:113, 1:29, :] = (z[:, :, 0:28, 64 * q:64 * q + 64]
                                       + b_ref[...])
    m = []
    for q in range(4):
        a = s_ref[:, q, 0:111:2, :, :]
        b = s_ref[:, q, 1:112:2, :, :]
        c = s_ref[:, q, 2:113:2, :, :]
        m.append(jnp.maximum(jnp.maximum(a, b), c))
    p0 = jnp.maximum(jnp.maximum(m[3][:, :, 0:28], m[0][:, :, 1:29]),
                     m[1][:, :, 1:29])
    p1 = jnp.maximum(jnp.maximum(m[1][:, :, 1:29], m[2][:, :, 1:29]),
                     m[3][:, :, 1:29])
    p0 = jnp.maximum(p0, 0.0)
    p1 = jnp.maximum(p1, 0.0)
    out_ref[...] = jnp.zeros((B, 58, 64, 64), _F32)
    out_ref[:, 1:57, 1:57:2, :] = p0
    out_ref[:, 1:57, 2:58:2, :] = p1


_BS = 1


def _stem_call(x8, w, b):
    return pl.pallas_call(
        functools.partial(_stem_kernel, B=_BS),
        grid=(_N // _BS,),
        in_specs=[
            pl.BlockSpec((_BS, 2, 116, 40, 8), lambda n: (n, 0, 0, 0, 0)),
            pl.BlockSpec((112, 256), lambda n: (0, 0)),
            pl.BlockSpec((1, 64), lambda n: (0, 0)),
        ],
        out_specs=pl.BlockSpec((_BS, 58, 64, 64), lambda n: (n, 0, 0, 0)),
        out_shape=jax.ShapeDtypeStruct((_N, 58, 64, 64), _F32),
        scratch_shapes=[pltpu.VMEM((_BS, 4, 114, 32, 64), _F32)],
    )(x8, w, b)


def _plain_block_kernel(x_ref, w1_ref, b1_ref, w2_ref, b2_ref, out_ref,
                        hp_ref, *, B, H, W, Wb, C):
    xp = x_ref[...]
    h = _conv3x3_mm(xp[:, 0:H], xp[:, 1:H + 1], xp[:, 2:H + 2], w1_ref[...],
                    B, H, W, Wb, C) + b1_ref[...]
    h = jnp.maximum(h, 0.0)
    hp_ref[...] = jnp.zeros((B, H + 2, Wb, C), _F32)
    hp_ref[:, 1:H + 1, 1:W + 1, :] = h
    o = _conv3x3_mm(hp_ref[:, 0:H], hp_ref[:, 1:H + 1], hp_ref[:, 2:H + 2],
                    w2_ref[...], B, H, W, Wb, C) + b2_ref[...]
    o = jnp.maximum(o + xp[:, 1:H + 1, 1:W + 1, :], 0.0)
    _store_padded(out_ref, o, B, H, W, Wb, C)


def _plain_block_call(buf, w1, b1, w2, b2, B, H, W, Wb, C):
    return pl.pallas_call(
        functools.partial(_plain_block_kernel, B=B, H=H, W=W, Wb=Wb, C=C),
        grid=(_N // B,),
        in_specs=[
            pl.BlockSpec((B, H + 2, Wb, C), lambda n: (n, 0, 0, 0)),
            pl.BlockSpec((3 * C, 3 * C), lambda n: (0, 0)),
            pl.BlockSpec((1, C), lambda n: (0, 0)),
            pl.BlockSpec((3 * C, 3 * C), lambda n: (0, 0)),
            pl.BlockSpec((1, C), lambda n: (0, 0)),
        ],
        out_specs=pl.BlockSpec((B, H + 2, Wb, C), lambda n: (n, 0, 0, 0)),
        out_shape=jax.ShapeDtypeStruct((_N, H + 2, Wb, C), _F32),
        scratch_shapes=[pltpu.VMEM((B, H + 2, Wb, C), _F32)],
    )(buf, w1, b1, w2, b2)


def _down_block_kernel(p_ref, w1_ref, b1_ref, w2_ref, b2_ref,
                       wd_ref, bd_ref, out_ref, hp_ref,
                       *, B, Ho, Wo, Wob, Wb2, Cin, Cout):
    taps = [p_ref[:, 2 * (ky % 2) + (kx % 2),
                  ky // 2:ky // 2 + Ho, kx // 2:kx // 2 + Wo, :]
            for ky in range(3) for kx in range(3)]
    t = jnp.concatenate(taps, axis=-1)
    if Wob > Wo:
        t = jnp.concatenate(
            [t, jnp.zeros((B, Ho, Wob - Wo, 9 * Cin), _F32)], axis=2)
    h = jnp.dot(t.reshape(B * Ho * Wob, 9 * Cin), w1_ref[...],
                preferred_element_type=_F32).reshape(B, Ho, Wob, Cout)
    h = jnp.maximum(h + b1_ref[...], 0.0)[:, :, 0:Wo, :]
    hp_ref[...] = jnp.zeros((B, Ho + 2, Wb2, Cout), _F32)
    hp_ref[:, 1:Ho + 1, 1:Wo + 1, :] = h
    o = _conv3x3_mm(hp_ref[:, 0:Ho], hp_ref[:, 1:Ho + 1], hp_ref[:, 2:Ho + 2],
                    w2_ref[...], B, Ho, Wo, Wb2, Cout) + b2_ref[...]
    xd = p_ref[:, 3, 0:Ho, 0:Wo, :]
    if Wob > Wo:
        xd = jnp.concatenate(
            [xd, jnp.zeros((B, Ho, Wob - Wo, Cin), _F32)], axis=2)
    res = jnp.dot(xd.reshape(B * Ho * Wob, Cin), wd_ref[...],
                  preferred_element_type=_F32).reshape(B, Ho, Wob, Cout)
    res = res[:, :, 0:Wo, :] + bd_ref[...]
    o = jnp.maximum(o + res, 0.0)
    _store_padded(out_ref, o, B, Ho, Wo, Wb2, Cout)


def _parity4(buf):
    N, Hp, Wb, C = buf.shape
    v = buf.reshape(N, Hp // 2, 2, Wb // 2, 2, C)
    v = jnp.transpose(v, (0, 2, 4, 1, 3, 5))
    return v.reshape(N, 4, Hp // 2, Wb // 2, C)


def _down_block_call(pt, w1, b1, w2, b2, wd, bd,
                     B, Ho, Wo, Wob, Wb2, Cin, Cout):
    _, _, Hq, Wq, _ = pt.shape
    return pl.pallas_call(
        functools.partial(_down_block_kernel, B=B, Ho=Ho, Wo=Wo, Wob=Wob,
                          Wb2=Wb2, Cin=Cin, Cout=Cout),
        grid=(_N // B,),
        in_specs=[
            pl.BlockSpec((B, 4, Hq, Wq, Cin), lambda n: (n, 0, 0, 0, 0)),
            pl.BlockSpec((9 * Cin, Cout), lambda n: (0, 0)),
            pl.BlockSpec((1, Cout), lambda n: (0, 0)),
            pl.BlockSpec((3 * Cout, 3 * Cout), lambda n: (0, 0)),
            pl.BlockSpec((1, Cout), lambda n: (0, 0)),
            pl.BlockSpec((Cin, Cout), lambda n: (0, 0)),
            pl.BlockSpec((1, Cout), lambda n: (0, 0)),
        ],
        out_specs=pl.BlockSpec((B, Ho + 2, Wb2, Cout), lambda n: (n, 0, 0, 0)),
        out_shape=jax.ShapeDtypeStruct((_N, Ho + 2, Wb2, Cout), _F32),
        scratch_shapes=[pltpu.VMEM((B, Ho + 2, Wb2, Cout), _F32)],
    )(pt, w1, b1, w2, b2, wd, bd)


def _head_kernel(x_ref, cw_ref, cb_ref, pw_ref, pb_ref, idx_ref, probs_ref):
    x = x_ref[...]
    feat = jnp.sum(x, axis=(1, 2)) * (1.0 / 49.0)
    cl = jnp.dot(feat, cw_ref[...], preferred_element_type=_F32) + cb_ref[...]
    pll = jnp.dot(feat, pw_ref[...], preferred_element_type=_F32) + pb_ref[...]

    def smax(l):
        m = jnp.max(l, axis=1, keepdims=True)
        e = jnp.exp(l - m)
        return e / jnp.sum(e, axis=1, keepdims=True)

    cp = smax(cl)
    pp = smax(pll)
    cmax = jnp.max(cp, axis=1, keepdims=True)
    pmax = jnp.max(pp, axis=1, keepdims=True)
    ciota = lax.broadcasted_iota(jnp.int32, (_N, 6), 1)
    piota = lax.broadcasted_iota(jnp.int32, (_N, 2), 1)
    cidx = jnp.min(jnp.where(cp == cmax, ciota, 6), axis=1, keepdims=True)
    pidx = jnp.min(jnp.where(pp == pmax, piota, 2), axis=1, keepdims=True)
    wp = 0.3 * pmax
    wc = 0.7 * cmax
    tot = wp + wc + 1e-8
    idx_ref[...] = jnp.concatenate([pidx, cidx + 2], axis=1)
    probs_ref[...] = jnp.concatenate([wp / tot, wc / tot], axis=1)


def _head_call(buf4, cw, cb, pw, pb):
    return pl.pallas_call(
        _head_kernel,
        out_shape=[jax.ShapeDtypeStruct((_N, 2), jnp.int32),
                   jax.ShapeDtypeStruct((_N, 2), _F32)],
    )(buf4, cw, cb, pw, pb)


def _w3x3(w, bn):
    w, b = _fold_bn(w, bn)
    wt = jnp.transpose(w, (2, 1, 3, 0))
    C, Cout = w.shape[1], w.shape[0]
    return wt.reshape(3 * C, 3 * Cout), b.reshape(1, Cout)


def _w3x3s2(w, bn):
    w, b = _fold_bn(w, bn)
    wt = jnp.transpose(w, (2, 3, 1, 0))
    return wt.reshape(9 * w.shape[1], w.shape[0]), b.reshape(1, w.shape[0])


def _w1x1(w, bn):
    w, b = _fold_bn(w, bn)
    return jnp.transpose(w[:, :, 0, 0], (1, 0)), b.reshape(1, w.shape[0])


def _wstem(w, bn):
    w, b = _fold_bn(w, bn)
    wk = w[:, 0]
    z = jnp.zeros((64,), _F32)
    qmats = []
    for q in range(4):
        rows = []
        for ky in range(7):
            for dj in range(2):
                for l in range(8):
                    kx = 8 * dj + l - 2 * q
                    rows.append(wk[:, ky, kx] if 0 <= kx < 7 else z)
        qmats.append(jnp.stack(rows, axis=0))
    return jnp.concatenate(qmats, axis=1), b.reshape(1, 64)


_GEOM = [
    (56, 56, 64), (28, 28, 32), (14, 14, 16), (7, 7, 16),
]
_CH = [64, 128, 256, 512]
_BB = [2, 4, 8, 8]


def kernel(x, params):
    xp = jnp.pad(x[:, 0], ((0, 0), (3, 5), (3, 85)))
    xp = jnp.pad(xp, ((0, 0), (0, 0), (0, 8)))
    x8 = xp.reshape(_N, 116, 2, 40, 8)
    x8 = jnp.transpose(x8, (0, 2, 1, 3, 4))
    ws, bs = _wstem(params['conv1'], params['bn1'])
    buf = _stem_call(x8, ws, bs)

    for li, layer in enumerate(params['layers']):
        H, W, Wb = _GEOM[li]
        C = _CH[li]
        B = _BB[li]
        blk0, blk1 = layer
        if li == 0:
            w1, b1 = _w3x3(blk0['conv1'], blk0['bn1'])
            w2, b2 = _w3x3(blk0['conv2'], blk0['bn2'])
            buf = _plain_block_call(buf, w1, b1, w2, b2, B, H, W, Wb, C)
        else:
            Cin = _CH[li - 1]
            Wob = 8 * ((W + 7) // 8)
            w1, b1 = _w3x3s2(blk0['conv1'], blk0['bn1'])
            w2, b2 = _w3x3(blk0['conv2'], blk0['bn2'])
            wd, bd = _w1x1(blk0['down'], blk0['dbn'])
            buf = _down_block_call(_parity4(buf), w1, b1, w2, b2, wd, bd,
                                   B, H, W, Wob, Wb, Cin, C)
        w1, b1 = _w3x3(blk1['conv1'], blk1['bn1'])
        w2, b2 = _w3x3(blk1['conv2'], blk1['bn2'])
        buf = _plain_block_call(buf, w1, b1, w2, b2, B, H, W, Wb, C)

    idx, probs = _head_call(
        buf,
        jnp.transpose(params['child_w'], (1, 0)),
        params['child_b'].reshape(1, 6),
        jnp.transpose(params['parent_w'], (1, 0)),
        params['parent_b'].reshape(1, 2),
    )
    return (idx, probs)
